# baseline (speedup 1.0000x reference)
import jax
import jax.numpy as jnp
import numpy as np
from jax import lax
from jax.experimental import pallas as pl
from jax.experimental.pallas import tpu as pltpu
from jax.experimental.pallas import tpu_sc as plsc

C = 96
H = 512
W = 512
FH = 128
FW = 128
L = FH * FW
NW = 32
PR_PER_W = FH // NW
ROWS_PER_W = 4 * PR_PER_W
LW = PR_PER_W * FW
CG = 4
NB = 2 * CG

with jax.default_device(jax.devices("cpu")[0]):
    _SIDX = np.asarray(
        jax.random.randint(jax.random.key(42), (L,), 0, 16, jnp.int32))

_LL = np.arange(L)
_ROW = 4 * ((_LL // FW) % PR_PER_W) + (_SIDX >> 2)
_COL = 4 * (_LL % FW) + (_SIDX & 3)
_IBUF = ((_ROW << 9) | _COL).astype(np.int32)


def _slab_src(x_hbm, wid, c):
    return x_hbm.at[pl.ds(c * H + wid * ROWS_PER_W, ROWS_PER_W), :]


def _body(x_hbm, ib_hbm, out_hbm, ibuf, xbufs, obuf, load_sem,
          store_sem, sem_s):
    cid = lax.axis_index("c")
    sid = lax.axis_index("s")
    wid = sid * 2 + cid
    base_l = wid * LW

    pltpu.async_copy(ib_hbm.at[pl.ds(base_l, LW)], ibuf, sem_s)
    for b in range(NB):
        pltpu.async_copy(_slab_src(x_hbm, wid, b), xbufs[b], load_sem)
    pltpu.make_async_copy(ib_hbm.at[pl.ds(base_l, LW)], ibuf, sem_s).wait()

    def group(g, carry):
        for half in range(NB // CG):
            c0 = g * NB + half * CG
            for k in range(CG):
                pltpu.make_async_copy(
                    _slab_src(x_hbm, wid, c0 + k), xbufs[half * CG + k],
                    load_sem,
                ).wait()
            @plsc.parallel_loop(0, LW // 16, unroll=4)
            def _sweep(i):
                p = ibuf[pl.ds(i * 16, 16)]
                row = p >> 9
                col = p & (W - 1)
                for k in range(CG):
                    obuf[c0 + k, pl.ds(i * 16, 16)] = plsc.load_gather(
                        xbufs[half * CG + k], [row, col]
                    )

            @pl.when(c0 + NB + CG <= C)
            def _():
                for k in range(CG):
                    pltpu.async_copy(
                        _slab_src(x_hbm, wid, c0 + NB + k),
                        xbufs[half * CG + k], load_sem,
                    )

            pltpu.async_copy(
                obuf.at[pl.ds(c0, CG), :],
                out_hbm.at[pl.ds(c0, CG), pl.ds(base_l, LW)],
                store_sem,
            )

        return carry

    lax.fori_loop(0, C // NB, group, 0)
    for g in range(C // CG):
        pltpu.make_async_copy(
            obuf.at[pl.ds(g * CG, CG), :],
            out_hbm.at[pl.ds(g * CG, CG), pl.ds(base_l, LW)],
            store_sem,
        ).wait()


def _body_wrap(x_hbm, ib_hbm, out_hbm, ibuf, *rest):
    xbufs = rest[:NB]
    obuf, load_sem, store_sem, sem_s = rest[NB:]
    _body(x_hbm, ib_hbm, out_hbm, ibuf, xbufs, obuf, load_sem,
          store_sem, sem_s)


@jax.jit
def _run(xr, sidx):
    mesh = plsc.VectorSubcoreMesh(core_axis_name="c", subcore_axis_name="s")
    kfn = pl.kernel(
        _body_wrap,
        out_type=jax.ShapeDtypeStruct((C, L), jnp.float32),
        mesh=mesh,
        scratch_types=[
            pltpu.VMEM((LW,), jnp.int32),
        ] + [
            pltpu.VMEM((ROWS_PER_W, W), jnp.float32)
            for _ in range(NB)
        ] + [
            pltpu.VMEM((C, LW), jnp.float32),
            pltpu.SemaphoreType.DMA,
            pltpu.SemaphoreType.DMA,
            pltpu.SemaphoreType.DMA,
        ],
        compiler_params=pltpu.CompilerParams(needs_layout_passes=False),
    )
    return kfn(xr, sidx)


def kernel(x):
    b, c, h, w = x.shape
    sidx = jnp.asarray(_IBUF)
    xr = x.reshape(C * H, W)
    out = _run(xr, sidx)
    return out.reshape(1, C, L)
